# Initial kernel scaffold; baseline (speedup 1.0000x reference)
import jax
import jax.numpy as jnp
from jax.experimental import pallas as pl


def kernel(h, edges, edge_attr, emb_W, emb_b, e1W, e1b, e2W, e2b, n1W, n1b, n2W, n2b, d1W, d1b, d2W, d2b):
    raise NotImplementedError("write your pallas kernel here")


# speedup vs baseline: 3.0129x; 3.0129x over previous
import functools

import jax
import jax.numpy as jnp
from jax import lax
from jax.experimental import pallas as pl
from jax.experimental.pallas import tpu as pltpu
from jax.experimental.pallas import tpu_sc as plsc

N = 10000
E = 320000
D = 128
H = 128
NL = 4
DE = 2

NC = 2
NS = 16
NW = NC * NS
CH = 128
NCHUNK = E // CH
CPW = -(-NCHUNK // NW)

ZR = 80
NZC = N // ZR
ZPW = -(-NZC // NS)

BN = 2000
BE = 2000

_f32 = jnp.float32
_mesh = plsc.VectorSubcoreMesh(core_axis_name="c", subcore_axis_name="s")


def _dot(a, b):
    return jnp.dot(a, b, preferred_element_type=_f32)


def _silu(x):
    return x * jax.nn.sigmoid(x)


def _embed_body(h_ref, w_ref, b_ref, o_ref):
    o_ref[...] = _dot(h_ref[...], w_ref[...]) + b_ref[...]


def _xsxd_body(x_ref, ws_ref, wd_ref, o1_ref, o2_ref):
    x = x_ref[...]
    o1_ref[...] = _dot(x, ws_ref[...])
    o2_ref[...] = _dot(x, wd_ref[...])


def _edge_body(g1_ref, g2_ref, ea_ref, wea_ref, b1_ref, w2_ref, b2_ref, o_ref):
    ea = ea_ref[...]
    pre = (g1_ref[...] + g2_ref[...] + b1_ref[...]
           + ea[:, 0:1] * wea_ref[0:1, :] + ea[:, 1:2] * wea_ref[1:2, :])
    m = _silu(pre)
    o_ref[...] = _silu(_dot(m, w2_ref[...]) + b2_ref[...])


def _node_body(x_ref, p0_ref, p1_ref, wx_ref, wa_ref, b1_ref, w2_ref, b2_ref,
               o_ref):
    x = x_ref[...]
    agg = p0_ref[0] + p1_ref[0]
    t = _silu(_dot(x, wx_ref[...]) + _dot(agg, wa_ref[...]) + b1_ref[...])
    o_ref[...] = x + _dot(t, w2_ref[...]) + b2_ref[...]


def _dec_body(x_ref, w1_ref, b1_ref, w2_ref, b2_ref, o_ref):
    t = _silu(_dot(x_ref[...], w1_ref[...]) + b1_ref[...])
    o_ref[...] = _dot(t, w2_ref[...]) + b2_ref[...]


def _row_block(bn):
    return pl.BlockSpec((bn, H), lambda i: (i, 0))


def _full(shape):
    return pl.BlockSpec(shape, lambda i: tuple(0 for _ in shape))


def _embed(h, w, b):
    return pl.pallas_call(
        _embed_body,
        grid=(N // BN,),
        in_specs=[pl.BlockSpec((BN, D), lambda i: (i, 0)),
                  _full((D, H)), _full((1, H))],
        out_specs=_row_block(BN),
        out_shape=jax.ShapeDtypeStruct((N, H), _f32),
    )(h, w, b.reshape(1, H))


def _xsxd(x, ws, wd):
    return pl.pallas_call(
        _xsxd_body,
        grid=(N // BN,),
        in_specs=[_row_block(BN), _full((H, H)), _full((H, H))],
        out_specs=[_row_block(BN), _row_block(BN)],
        out_shape=[jax.ShapeDtypeStruct((N, H), _f32),
                   jax.ShapeDtypeStruct((N, H), _f32)],
    )(x, ws, wd)


def _edge_mlp(g1, g2, ea, wea, b1, w2, b2):
    return pl.pallas_call(
        _edge_body,
        grid=(E // BE,),
        in_specs=[_row_block(BE), _row_block(BE),
                  pl.BlockSpec((BE, DE), lambda i: (i, 0)),
                  _full((DE, H)), _full((1, H)), _full((H, H)), _full((1, H))],
        out_specs=_row_block(BE),
        out_shape=jax.ShapeDtypeStruct((E, H), _f32),
    )(g1, g2, ea, wea, b1.reshape(1, H), w2, b2.reshape(1, H))


def _node_mlp(x, parts, wx, wa, b1, w2, b2):
    return pl.pallas_call(
        _node_body,
        grid=(N // BN,),
        in_specs=[_row_block(BN),
                  pl.BlockSpec((1, BN, H), lambda i: (0, i, 0)),
                  pl.BlockSpec((1, BN, H), lambda i: (1, i, 0)),
                  _full((H, H)), _full((H, H)), _full((1, H)),
                  _full((H, H)), _full((1, H))],
        out_specs=_row_block(BN),
        out_shape=jax.ShapeDtypeStruct((N, H), _f32),
    )(x, parts, parts, wx, wa, b1.reshape(1, H), w2, b2.reshape(1, H))


def _decoder(x, w1, b1, w2, b2):
    w2p = jnp.zeros((H, 128), _f32).at[:, :3].set(w2)
    b2p = jnp.zeros((1, 128), _f32).at[0, :3].set(b2)
    y = pl.pallas_call(
        _dec_body,
        grid=(N // BN,),
        in_specs=[_row_block(BN), _full((H, H)), _full((1, H)),
                  _full((H, 128)), _full((1, 128))],
        out_specs=pl.BlockSpec((BN, 128), lambda i: (i, 0)),
        out_shape=jax.ShapeDtypeStruct((N, 128), _f32),
    )(x, w1, b1.reshape(1, H), w2p, b2p)
    return y[:, :3]


def _sc_gather_body(xs_hbm, xd_hbm, row_hbm, col_hbm, g1_hbm, g2_hbm,
                    idx1, idx2, buf1, buf2, sem1, sem2):
    wid = lax.axis_index("s") * NC + lax.axis_index("c")

    @pl.loop(0, CPW)
    def _(i):
        c = wid + i * NW

        @pl.when(c < NCHUNK)
        def _():
            base = c * CH
            pltpu.sync_copy(row_hbm.at[pl.ds(base, CH)], idx1)
            pltpu.sync_copy(col_hbm.at[pl.ds(base, CH)], idx2)
            cp1 = pltpu.async_copy(xs_hbm.at[idx1], buf1, sem1)
            cp2 = pltpu.async_copy(xd_hbm.at[idx2], buf2, sem2)
            cp1.wait()
            cp2.wait()
            pltpu.sync_copy(buf1, g1_hbm.at[pl.ds(base, CH)])
            pltpu.sync_copy(buf2, g2_hbm.at[pl.ds(base, CH)])


def _sc_gather(xs, xd, row, col):
    k = pl.kernel(
        _sc_gather_body,
        out_type=[jax.ShapeDtypeStruct((E, H), _f32),
                  jax.ShapeDtypeStruct((E, H), _f32)],
        mesh=_mesh,
        scratch_types=[pltpu.VMEM((CH,), jnp.int32),
                       pltpu.VMEM((CH,), jnp.int32),
                       pltpu.VMEM((CH, H), _f32),
                       pltpu.VMEM((CH, H), _f32),
                       pltpu.SemaphoreType.DMA,
                       pltpu.SemaphoreType.DMA],
    )
    return k(xs, xd, row, col)


def _sc_scatter_body(m2_hbm, row_hbm, out_hbm, idx, buf, zbuf, acc, sem):
    cid = lax.axis_index("c")
    sid = lax.axis_index("s")
    wid = sid * NC + cid

    @pl.loop(0, ZR)
    def _(r):
        @pl.loop(0, H // 16)
        def _(j):
            zbuf[r, pl.ds(j * 16, 16)] = jnp.zeros((16,), _f32)

    @pl.loop(0, ZPW)
    def _(j):
        k = sid + j * NS

        @pl.when(k < NZC)
        def _():
            pltpu.sync_copy(zbuf, acc.at[pl.ds(k * ZR, ZR)])

    plsc.subcore_barrier()

    @pl.loop(0, CPW)
    def _(i):
        c = wid + i * NW

        @pl.when(c < NCHUNK)
        def _():
            base = c * CH
            pltpu.sync_copy(row_hbm.at[pl.ds(base, CH)], idx)
            pltpu.sync_copy(m2_hbm.at[pl.ds(base, CH)], buf)
            pltpu.sync_copy(buf, acc.at[idx], add=True)

    plsc.subcore_barrier()

    @pl.loop(0, ZPW)
    def _(j):
        k = sid + j * NS

        @pl.when(k < NZC)
        def _():
            r0 = k * ZR
            pltpu.sync_copy(acc.at[pl.ds(r0, ZR)],
                            out_hbm.at[cid, pl.ds(r0, ZR)])


def _sc_scatter(m2, row):
    k = pl.kernel(
        _sc_scatter_body,
        out_type=jax.ShapeDtypeStruct((NC, N, H), _f32),
        mesh=_mesh,
        scratch_types=[pltpu.VMEM((CH,), jnp.int32),
                       pltpu.VMEM((CH, H), _f32),
                       pltpu.VMEM((ZR, H), _f32),
                       pltpu.VMEM_SHARED((N, H), _f32),
                       pltpu.SemaphoreType.DMA],
    )
    return k(m2, row)


def kernel(h, edges, edge_attr, emb_W, emb_b, e1W, e1b, e2W, e2b,
           n1W, n1b, n2W, n2b, d1W, d1b, d2W, d2b):
    row = edges[0]
    col = edges[1]
    x = _embed(h, emb_W, emb_b)
    for i in range(NL):
        ws = e1W[i, :H]
        wd = e1W[i, H:2 * H]
        wea = e1W[i, 2 * H:]
        xs, xd = _xsxd(x, ws, wd)
        g1, g2 = _sc_gather(xs, xd, row, col)
        m2 = _edge_mlp(g1, g2, edge_attr, wea, e1b[i], e2W[i], e2b[i])
        parts = _sc_scatter(m2, row)
        x = _node_mlp(x, parts, n1W[i, :H], n1W[i, H:], n1b[i],
                      n2W[i], n2b[i])
    return _decoder(x, d1W, d1b, d2W, d2b)


# speedup vs baseline: 4.2637x; 1.4151x over previous
import jax
import jax.numpy as jnp
from jax import lax
from jax.experimental import pallas as pl
from jax.experimental.pallas import tpu as pltpu
from jax.experimental.pallas import tpu_sc as plsc

N = 10000
E = 320000
D = 128
H = 128
HP = H // 2
NL = 4
DE = 2

NC = 2
NS = 16
NW = NC * NS
CH = 80
NCHUNK = E // CH
MAXC = 128
NCPAD = MAXC * NW

ZR = 80
NZC = N // ZR
ZPW = -(-NZC // NS)

BN = 2000
BE = 2000

_f32 = jnp.float32
_i32 = jnp.int32
_u32 = jnp.uint32
_mesh = plsc.VectorSubcoreMesh(core_axis_name="c", subcore_axis_name="s")


def _dot(a, b):
    return jnp.dot(a, b, preferred_element_type=_f32)


def _silu(x):
    return x * jax.nn.sigmoid(x)


def _pack_bf16(y):
    u = jax.lax.bitcast_convert_type(y, _u32)
    r = (u + jnp.uint32(0x7FFF) + ((u >> 16) & jnp.uint32(1))) >> 16
    packed = r[:, :HP] | (r[:, HP:] << 16)
    return jax.lax.bitcast_convert_type(packed, _i32)


def _unpack_f32(g):
    u = jax.lax.bitcast_convert_type(g, _u32)
    lo = jax.lax.bitcast_convert_type(u << 16, _f32)
    hi = jax.lax.bitcast_convert_type(u & jnp.uint32(0xFFFF0000), _f32)
    return lo, hi


def _embed_body(h_ref, w_ref, b_ref, o_ref):
    o_ref[...] = _dot(h_ref[...], w_ref[...]) + b_ref[...]


def _xsxd_body(x_ref, ws_ref, wd_ref, o1_ref, o2_ref):
    x = x_ref[...]
    o1_ref[...] = _dot(x, ws_ref[...])
    o2_ref[...] = _dot(x, wd_ref[...])


def _edge_body(g_ref, ea_ref, wea_ref, b1_ref, w2_ref, b2_ref, o_ref):
    ea = ea_ref[...]
    pre = (g_ref[...] + b1_ref[...]
           + ea[:, 0:1] * wea_ref[0:1, :] + ea[:, 1:2] * wea_ref[1:2, :])
    m = _silu(pre)
    o_ref[...] = _silu(_dot(m, w2_ref[...]) + b2_ref[...])


def _node_body(x_ref, p0_ref, p1_ref, wx_ref, wa_ref, b1_ref, w2_ref, b2_ref,
               o_ref):
    x = x_ref[...]
    agg = p0_ref[0] + p1_ref[0]
    t = _silu(_dot(x, wx_ref[...]) + _dot(agg, wa_ref[...]) + b1_ref[...])
    o_ref[...] = x + _dot(t, w2_ref[...]) + b2_ref[...]


def _dec_body(x_ref, w1_ref, b1_ref, w2_ref, b2_ref, o_ref):
    t = _silu(_dot(x_ref[...], w1_ref[...]) + b1_ref[...])
    o_ref[...] = _dot(t, w2_ref[...]) + b2_ref[...]


def _row_block(bn):
    return pl.BlockSpec((bn, H), lambda i: (i, 0))


def _full(shape):
    return pl.BlockSpec(shape, lambda i: tuple(0 for _ in shape))


def _embed(h, w, b):
    return pl.pallas_call(
        _embed_body,
        grid=(N // BN,),
        in_specs=[pl.BlockSpec((BN, D), lambda i: (i, 0)),
                  _full((D, H)), _full((1, H))],
        out_specs=_row_block(BN),
        out_shape=jax.ShapeDtypeStruct((N, H), _f32),
    )(h, w, b.reshape(1, H))


def _xsxd(x, ws, wd):
    return pl.pallas_call(
        _xsxd_body,
        grid=(N // BN,),
        in_specs=[_row_block(BN), _full((H, H)), _full((H, H))],
        out_specs=[_row_block(BN), _row_block(BN)],
        out_shape=[jax.ShapeDtypeStruct((N, H), _f32),
                   jax.ShapeDtypeStruct((N, H), _f32)],
    )(x, ws, wd)


def _edge_mlp(g, ea, wea, b1, w2, b2):
    return pl.pallas_call(
        _edge_body,
        grid=(E // BE,),
        in_specs=[_row_block(BE),
                  pl.BlockSpec((BE, DE), lambda i: (i, 0)),
                  _full((DE, H)), _full((1, H)), _full((H, H)), _full((1, H))],
        out_specs=_row_block(BE),
        out_shape=jax.ShapeDtypeStruct((E, H), _f32),
    )(g, ea, wea, b1.reshape(1, H), w2, b2.reshape(1, H))


def _node_mlp(x, parts, wx, wa, b1, w2, b2):
    return pl.pallas_call(
        _node_body,
        grid=(N // BN,),
        in_specs=[_row_block(BN),
                  pl.BlockSpec((1, BN, H), lambda i: (0, i, 0)),
                  pl.BlockSpec((1, BN, H), lambda i: (1, i, 0)),
                  _full((H, H)), _full((H, H)), _full((1, H)),
                  _full((H, H)), _full((1, H))],
        out_specs=_row_block(BN),
        out_shape=jax.ShapeDtypeStruct((N, H), _f32),
    )(x, parts, parts, wx, wa, b1.reshape(1, H), w2, b2.reshape(1, H))


def _decoder(x, w1, b1, w2, b2):
    w2p = jnp.zeros((H, 128), _f32).at[:, :3].set(w2)
    b2p = jnp.zeros((1, 128), _f32).at[0, :3].set(b2)
    y = pl.pallas_call(
        _dec_body,
        grid=(N // BN,),
        in_specs=[_row_block(BN), _full((H, H)), _full((1, H)),
                  _full((H, 128)), _full((1, 128))],
        out_specs=pl.BlockSpec((BN, 128), lambda i: (i, 0)),
        out_shape=jax.ShapeDtypeStruct((N, 128), _f32),
    )(x, w1, b1.reshape(1, H), w2p, b2p)
    return y[:, :3]


def _worker_range(wid):
    c0 = wid * MAXC
    cnt = jnp.minimum(MAXC, NCHUNK - c0)
    return c0, cnt


def _sc_gather_body(xs_hbm, xd_hbm, row_hbm, col_hbm, g_hbm,
                    ridx, cidx, ident, b1a, b1b, b2a, b2b, shared,
                    s1a, s1b, s2a, s2b):
    cid = lax.axis_index("c")
    sid = lax.axis_index("s")
    wid = sid * NC + cid
    c0, cnt = _worker_range(wid)
    pltpu.sync_copy(row_hbm.at[pl.ds(c0, MAXC)], ridx)
    pltpu.sync_copy(col_hbm.at[pl.ds(c0, MAXC)], cidx)

    @pl.loop(0, CH // 16)
    def _(j):
        ia = lax.broadcasted_iota(_i32, (16,), 0) + j * 16
        ident[pl.ds(j * 16, 16)] = ia + sid * CH

    def fire(k, buf1, buf2, sem1, sem2):
        @pl.when(k < cnt)
        def _():
            pltpu.make_async_copy(xs_hbm.at[ridx.at[k]], buf1, sem1).start()
            pltpu.make_async_copy(xd_hbm.at[cidx.at[k]], buf2, sem2).start()

    def drain(k, buf1, buf2, sem1, sem2):
        @pl.when(k < cnt)
        def _():
            base = (c0 + k) * CH
            r0 = sid * CH
            pltpu.make_async_copy(xs_hbm.at[ridx.at[k]], buf1, sem1).wait()
            pltpu.make_async_copy(xd_hbm.at[cidx.at[k]], buf2, sem2).wait()
            pltpu.sync_copy(buf1, shared.at[pl.ds(r0, CH)])
            pltpu.sync_copy(buf2, shared.at[ident], add=True)
            pltpu.sync_copy(shared.at[pl.ds(r0, CH)], g_hbm.at[pl.ds(base, CH)])

    fire(0, b1a, b2a, s1a, s2a)

    @pl.loop(0, (MAXC + 1) // 2)
    def _(j):
        k = 2 * j
        fire(k + 1, b1b, b2b, s1b, s2b)
        drain(k, b1a, b2a, s1a, s2a)
        fire(k + 2, b1a, b2a, s1a, s2a)
        drain(k + 1, b1b, b2b, s1b, s2b)


def _sc_gather(xs, xd, row2d, col2d):
    k = pl.kernel(
        _sc_gather_body,
        out_type=jax.ShapeDtypeStruct((E, H), _f32),
        mesh=_mesh,
        scratch_types=[pltpu.VMEM((MAXC, CH), _i32),
                       pltpu.VMEM((MAXC, CH), _i32),
                       pltpu.VMEM((CH,), _i32),
                       pltpu.VMEM((CH, H), _f32),
                       pltpu.VMEM((CH, H), _f32),
                       pltpu.VMEM((CH, H), _f32),
                       pltpu.VMEM((CH, H), _f32),
                       pltpu.VMEM_SHARED((NS * CH, H), _f32),
                       pltpu.SemaphoreType.DMA,
                       pltpu.SemaphoreType.DMA,
                       pltpu.SemaphoreType.DMA,
                       pltpu.SemaphoreType.DMA],
    )
    return k(xs, xd, row2d, col2d)


def _sc_scatter_body(m2_hbm, row_hbm, out_hbm, ridx, bufa, bufb, zbuf, acc,
                     sema, semb):
    cid = lax.axis_index("c")
    sid = lax.axis_index("s")
    wid = sid * NC + cid
    c0, cnt = _worker_range(wid)

    pltpu.sync_copy(row_hbm.at[pl.ds(c0, MAXC)], ridx)

    @pl.loop(0, ZR)
    def _(r):
        @pl.loop(0, H // 16)
        def _(j):
            zbuf[r, pl.ds(j * 16, 16)] = jnp.zeros((16,), _f32)

    @pl.loop(0, ZPW)
    def _(j):
        z = sid + j * NS

        @pl.when(z < NZC)
        def _():
            pltpu.sync_copy(zbuf, acc.at[pl.ds(z * ZR, ZR)])

    plsc.subcore_barrier()

    def fire(k, buf, sem):
        @pl.when(k < cnt)
        def _():
            base = (c0 + k) * CH
            pltpu.make_async_copy(m2_hbm.at[pl.ds(base, CH)], buf, sem).start()

    def drain(k, buf, sem):
        @pl.when(k < cnt)
        def _():
            base = (c0 + k) * CH
            pltpu.make_async_copy(m2_hbm.at[pl.ds(base, CH)], buf, sem).wait()
            pltpu.sync_copy(buf, acc.at[ridx.at[k]], add=True)

    fire(0, bufa, sema)

    @pl.loop(0, (MAXC + 1) // 2)
    def _(j):
        k = 2 * j
        fire(k + 1, bufb, semb)
        drain(k, bufa, sema)
        fire(k + 2, bufa, sema)
        drain(k + 1, bufb, semb)

    plsc.subcore_barrier()

    @pl.loop(0, ZPW)
    def _(j):
        z = sid + j * NS

        @pl.when(z < NZC)
        def _():
            r0 = z * ZR
            pltpu.sync_copy(acc.at[pl.ds(r0, ZR)],
                            out_hbm.at[cid, pl.ds(r0, ZR)])


def _sc_scatter(m2, row2d):
    k = pl.kernel(
        _sc_scatter_body,
        out_type=jax.ShapeDtypeStruct((NC, N, H), _f32),
        mesh=_mesh,
        scratch_types=[pltpu.VMEM((MAXC, CH), _i32),
                       pltpu.VMEM((CH, H), _f32),
                       pltpu.VMEM((CH, H), _f32),
                       pltpu.VMEM((ZR, H), _f32),
                       pltpu.VMEM_SHARED((N, H), _f32),
                       pltpu.SemaphoreType.DMA,
                       pltpu.SemaphoreType.DMA],
    )
    return k(m2, row2d)


def kernel(h, edges, edge_attr, emb_W, emb_b, e1W, e1b, e2W, e2b,
           n1W, n1b, n2W, n2b, d1W, d1b, d2W, d2b):
    pad = ((0, NCPAD - NCHUNK), (0, 0))
    row2d = jnp.pad(edges[0].reshape(NCHUNK, CH), pad)
    col2d = jnp.pad(edges[1].reshape(NCHUNK, CH), pad)
    x = _embed(h, emb_W, emb_b)
    for i in range(NL):
        ws = e1W[i, :H]
        wd = e1W[i, H:2 * H]
        wea = e1W[i, 2 * H:]
        xs, xd = _xsxd(x, ws, wd)
        g = _sc_gather(xs, xd, row2d, col2d)
        m2 = _edge_mlp(g, edge_attr, wea, e1b[i], e2W[i], e2b[i])
        parts = _sc_scatter(m2, row2d)
        x = _node_mlp(x, parts, n1W[i, :H], n1W[i, H:], n1b[i],
                      n2W[i], n2b[i])
    return _decoder(x, d1W, d1b, d2W, d2b)


# speedup vs baseline: 4.2861x; 1.0052x over previous
import jax
import jax.numpy as jnp
from jax import lax
from jax.experimental import pallas as pl
from jax.experimental.pallas import tpu as pltpu
from jax.experimental.pallas import tpu_sc as plsc

N = 10000
E = 320000
D = 128
H = 128
HP = H // 2
NL = 4
DE = 2

NC = 2
NS = 16
NW = NC * NS
CH = 80
NCHUNK = E // CH
MAXC = 128
NCPAD = MAXC * NW

ZR = 80
NZC = N // ZR
ZPW = -(-NZC // NS)

BN = 2000
BE = 2000

_f32 = jnp.float32
_i32 = jnp.int32
_u32 = jnp.uint32
_mesh = plsc.VectorSubcoreMesh(core_axis_name="c", subcore_axis_name="s")


def _dot(a, b):
    return jnp.dot(a, b, preferred_element_type=_f32)


def _silu(x):
    return x * (0.5 + 0.5 * jnp.tanh(0.5 * x))


def _pack_bf16(y):
    u = jax.lax.bitcast_convert_type(y, _u32)
    r = (u + jnp.uint32(0x7FFF) + ((u >> 16) & jnp.uint32(1))) >> 16
    packed = r[:, :HP] | (r[:, HP:] << 16)
    return jax.lax.bitcast_convert_type(packed, _i32)


def _unpack_f32(g):
    u = jax.lax.bitcast_convert_type(g, _u32)
    lo = jax.lax.bitcast_convert_type(u << 16, _f32)
    hi = jax.lax.bitcast_convert_type(u & jnp.uint32(0xFFFF0000), _f32)
    return lo, hi


def _embed_body(h_ref, w_ref, b_ref, o_ref):
    o_ref[...] = _dot(h_ref[...], w_ref[...]) + b_ref[...]


def _xsxd_body(x_ref, ws_ref, wd_ref, o1_ref, o2_ref):
    x = x_ref[...]
    o1_ref[...] = _dot(x, ws_ref[...])
    o2_ref[...] = _dot(x, wd_ref[...])


def _edge_body(g_ref, ea_ref, wea_ref, b1_ref, w2_ref, b2_ref, o_ref):
    ea = ea_ref[...]
    pre = (g_ref[...] + b1_ref[...]
           + ea[:, 0:1] * wea_ref[0:1, :] + ea[:, 1:2] * wea_ref[1:2, :])
    m = _silu(pre)
    o_ref[...] = _silu(_dot(m, w2_ref[...]) + b2_ref[...])


def _node_body(x_ref, p0_ref, p1_ref, wx_ref, wa_ref, b1_ref, w2_ref, b2_ref,
               o_ref):
    x = x_ref[...]
    agg = p0_ref[0] + p1_ref[0]
    t = _silu(_dot(x, wx_ref[...]) + _dot(agg, wa_ref[...]) + b1_ref[...])
    o_ref[...] = x + _dot(t, w2_ref[...]) + b2_ref[...]


def _dec_body(x_ref, w1_ref, b1_ref, w2_ref, b2_ref, o_ref):
    t = _silu(_dot(x_ref[...], w1_ref[...]) + b1_ref[...])
    o_ref[...] = _dot(t, w2_ref[...]) + b2_ref[...]


def _row_block(bn):
    return pl.BlockSpec((bn, H), lambda i: (i, 0))


def _full(shape):
    return pl.BlockSpec(shape, lambda i: tuple(0 for _ in shape))


def _embed(h, w, b):
    return pl.pallas_call(
        _embed_body,
        grid=(N // BN,),
        in_specs=[pl.BlockSpec((BN, D), lambda i: (i, 0)),
                  _full((D, H)), _full((1, H))],
        out_specs=_row_block(BN),
        out_shape=jax.ShapeDtypeStruct((N, H), _f32),
    )(h, w, b.reshape(1, H))


def _xsxd(x, ws, wd):
    return pl.pallas_call(
        _xsxd_body,
        grid=(N // BN,),
        in_specs=[_row_block(BN), _full((H, H)), _full((H, H))],
        out_specs=[_row_block(BN), _row_block(BN)],
        out_shape=[jax.ShapeDtypeStruct((N, H), _f32),
                   jax.ShapeDtypeStruct((N, H), _f32)],
    )(x, ws, wd)


def _edge_mlp(g, ea, wea, b1, w2, b2):
    return pl.pallas_call(
        _edge_body,
        grid=(E // BE,),
        in_specs=[_row_block(BE),
                  pl.BlockSpec((BE, DE), lambda i: (i, 0)),
                  _full((DE, H)), _full((1, H)), _full((H, H)), _full((1, H))],
        out_specs=_row_block(BE),
        out_shape=jax.ShapeDtypeStruct((E, H), _f32),
    )(g, ea, wea, b1.reshape(1, H), w2, b2.reshape(1, H))


def _node_mlp(x, parts, wx, wa, b1, w2, b2):
    return pl.pallas_call(
        _node_body,
        grid=(N // BN,),
        in_specs=[_row_block(BN),
                  pl.BlockSpec((1, BN, H), lambda i: (0, i, 0)),
                  pl.BlockSpec((1, BN, H), lambda i: (1, i, 0)),
                  _full((H, H)), _full((H, H)), _full((1, H)),
                  _full((H, H)), _full((1, H))],
        out_specs=_row_block(BN),
        out_shape=jax.ShapeDtypeStruct((N, H), _f32),
    )(x, parts, parts, wx, wa, b1.reshape(1, H), w2, b2.reshape(1, H))


def _decoder(x, w1, b1, w2, b2):
    w2p = jnp.zeros((H, 128), _f32).at[:, :3].set(w2)
    b2p = jnp.zeros((1, 128), _f32).at[0, :3].set(b2)
    y = pl.pallas_call(
        _dec_body,
        grid=(N // BN,),
        in_specs=[_row_block(BN), _full((H, H)), _full((1, H)),
                  _full((H, 128)), _full((1, 128))],
        out_specs=pl.BlockSpec((BN, 128), lambda i: (i, 0)),
        out_shape=jax.ShapeDtypeStruct((N, 128), _f32),
    )(x, w1, b1.reshape(1, H), w2p, b2p)
    return y[:, :3]


def _worker_range(wid):
    c0 = wid * MAXC
    cnt = jnp.minimum(MAXC, NCHUNK - c0)
    return c0, cnt


def _sc_gather_body(xs_hbm, xd_hbm, row_hbm, col_hbm, g_hbm,
                    ridx, cidx, ident, b1a, b1b, b2a, b2b, shared,
                    s1a, s1b, s2a, s2b):
    cid = lax.axis_index("c")
    sid = lax.axis_index("s")
    wid = sid * NC + cid
    c0, cnt = _worker_range(wid)
    pltpu.sync_copy(row_hbm.at[pl.ds(c0, MAXC)], ridx)
    pltpu.sync_copy(col_hbm.at[pl.ds(c0, MAXC)], cidx)

    @pl.loop(0, CH // 16)
    def _(j):
        ia = lax.broadcasted_iota(_i32, (16,), 0) + j * 16
        ident[pl.ds(j * 16, 16)] = ia + sid * CH

    def fire(k, buf1, buf2, sem1, sem2):
        @pl.when(k < cnt)
        def _():
            pltpu.make_async_copy(xs_hbm.at[ridx.at[k]], buf1, sem1).start()
            pltpu.make_async_copy(xd_hbm.at[cidx.at[k]], buf2, sem2).start()

    def drain(k, buf1, buf2, sem1, sem2):
        @pl.when(k < cnt)
        def _():
            base = (c0 + k) * CH
            r0 = sid * CH
            pltpu.make_async_copy(xs_hbm.at[ridx.at[k]], buf1, sem1).wait()
            pltpu.make_async_copy(xd_hbm.at[cidx.at[k]], buf2, sem2).wait()
            pltpu.sync_copy(buf1, shared.at[pl.ds(r0, CH)])
            pltpu.sync_copy(buf2, shared.at[ident], add=True)
            pltpu.sync_copy(shared.at[pl.ds(r0, CH)], g_hbm.at[pl.ds(base, CH)])

    fire(0, b1a, b2a, s1a, s2a)

    @pl.loop(0, (MAXC + 1) // 2)
    def _(j):
        k = 2 * j
        fire(k + 1, b1b, b2b, s1b, s2b)
        drain(k, b1a, b2a, s1a, s2a)
        fire(k + 2, b1a, b2a, s1a, s2a)
        drain(k + 1, b1b, b2b, s1b, s2b)


def _sc_gather(xs, xd, row2d, col2d):
    k = pl.kernel(
        _sc_gather_body,
        out_type=jax.ShapeDtypeStruct((E, H), _f32),
        mesh=_mesh,
        scratch_types=[pltpu.VMEM((MAXC, CH), _i32),
                       pltpu.VMEM((MAXC, CH), _i32),
                       pltpu.VMEM((CH,), _i32),
                       pltpu.VMEM((CH, H), _f32),
                       pltpu.VMEM((CH, H), _f32),
                       pltpu.VMEM((CH, H), _f32),
                       pltpu.VMEM((CH, H), _f32),
                       pltpu.VMEM_SHARED((NS * CH, H), _f32),
                       pltpu.SemaphoreType.DMA,
                       pltpu.SemaphoreType.DMA,
                       pltpu.SemaphoreType.DMA,
                       pltpu.SemaphoreType.DMA],
    )
    return k(xs, xd, row2d, col2d)


def _sc_scatter_body(m2_hbm, row_hbm, out_hbm, ridx, bufa, bufb, zbuf, acc,
                     sema, semb):
    cid = lax.axis_index("c")
    sid = lax.axis_index("s")
    wid = sid * NC + cid
    c0, cnt = _worker_range(wid)

    pltpu.sync_copy(row_hbm.at[pl.ds(c0, MAXC)], ridx)

    @pl.loop(0, ZR)
    def _(r):
        @pl.loop(0, H // 16)
        def _(j):
            zbuf[r, pl.ds(j * 16, 16)] = jnp.zeros((16,), _f32)

    @pl.loop(0, ZPW)
    def _(j):
        z = sid + j * NS

        @pl.when(z < NZC)
        def _():
            pltpu.sync_copy(zbuf, acc.at[pl.ds(z * ZR, ZR)])

    plsc.subcore_barrier()

    def fire(k, buf, sem):
        @pl.when(k < cnt)
        def _():
            base = (c0 + k) * CH
            pltpu.make_async_copy(m2_hbm.at[pl.ds(base, CH)], buf, sem).start()

    def drain(k, buf, sem):
        @pl.when(k < cnt)
        def _():
            base = (c0 + k) * CH
            pltpu.make_async_copy(m2_hbm.at[pl.ds(base, CH)], buf, sem).wait()
            pltpu.sync_copy(buf, acc.at[ridx.at[k]], add=True)

    fire(0, bufa, sema)

    @pl.loop(0, (MAXC + 1) // 2)
    def _(j):
        k = 2 * j
        fire(k + 1, bufb, semb)
        drain(k, bufa, sema)
        fire(k + 2, bufa, sema)
        drain(k + 1, bufb, semb)

    plsc.subcore_barrier()

    @pl.loop(0, ZPW)
    def _(j):
        z = sid + j * NS

        @pl.when(z < NZC)
        def _():
            r0 = z * ZR
            pltpu.sync_copy(acc.at[pl.ds(r0, ZR)],
                            out_hbm.at[cid, pl.ds(r0, ZR)])


def _sc_scatter(m2, row2d):
    k = pl.kernel(
        _sc_scatter_body,
        out_type=jax.ShapeDtypeStruct((NC, N, H), _f32),
        mesh=_mesh,
        scratch_types=[pltpu.VMEM((MAXC, CH), _i32),
                       pltpu.VMEM((CH, H), _f32),
                       pltpu.VMEM((CH, H), _f32),
                       pltpu.VMEM((ZR, H), _f32),
                       pltpu.VMEM_SHARED((N, H), _f32),
                       pltpu.SemaphoreType.DMA,
                       pltpu.SemaphoreType.DMA],
    )
    return k(m2, row2d)


def kernel(h, edges, edge_attr, emb_W, emb_b, e1W, e1b, e2W, e2b,
           n1W, n1b, n2W, n2b, d1W, d1b, d2W, d2b):
    pad = ((0, NCPAD - NCHUNK), (0, 0))
    row2d = jnp.pad(edges[0].reshape(NCHUNK, CH), pad)
    col2d = jnp.pad(edges[1].reshape(NCHUNK, CH), pad)
    x = _embed(h, emb_W, emb_b)
    for i in range(NL):
        ws = e1W[i, :H]
        wd = e1W[i, H:2 * H]
        wea = e1W[i, 2 * H:]
        xs, xd = _xsxd(x, ws, wd)
        g = _sc_gather(xs, xd, row2d, col2d)
        m2 = _edge_mlp(g, edge_attr, wea, e1b[i], e2W[i], e2b[i])
        parts = _sc_scatter(m2, row2d)
        x = _node_mlp(x, parts, n1W[i, :H], n1W[i, H:], n1b[i],
                      n2W[i], n2b[i])
    return _decoder(x, d1W, d1b, d2W, d2b)


# speedup vs baseline: 5.0884x; 1.1872x over previous
import jax
import jax.numpy as jnp
from jax import lax
from jax.experimental import pallas as pl
from jax.experimental.pallas import tpu as pltpu
from jax.experimental.pallas import tpu_sc as plsc

N = 10000
E = 320000
D = 128
H = 128
NL = 4
DE = 2

NSL = 2
ES = E // NSL

NC = 2
NS = 16
NW = NC * NS
CH = 80
NCHUNK = ES // CH
MAXC = 64
NCPAD = MAXC * NW

ZR = 80
NZC = N // ZR
ZPW = -(-NZC // NS)

BN = 2000
BE = 2000

_f32 = jnp.float32
_i32 = jnp.int32
_mesh = plsc.VectorSubcoreMesh(core_axis_name="c", subcore_axis_name="s")


def _dot(a, b):
    return jnp.dot(a, b, preferred_element_type=_f32)


def _silu(x):
    return x * (0.5 + 0.5 * jnp.tanh(0.5 * x))


def _embed_body(h_ref, w_ref, b_ref, ws_ref, wd_ref, o_ref, os_ref, od_ref):
    x = _dot(h_ref[...], w_ref[...]) + b_ref[...]
    o_ref[...] = x
    os_ref[...] = _dot(x, ws_ref[...])
    od_ref[...] = _dot(x, wd_ref[...])


def _edge_body(g_ref, ea_ref, wea_ref, b1_ref, w2_ref, b2_ref, o_ref):
    ea = ea_ref[...]
    pre = (g_ref[...] + b1_ref[...]
           + ea[:, 0:1] * wea_ref[0:1, :] + ea[:, 1:2] * wea_ref[1:2, :])
    m = _silu(pre)
    o_ref[...] = _silu(_dot(m, w2_ref[...]) + b2_ref[...])


def _node_mid_body(x_ref, pa_ref, pb_ref, wx_ref, wa_ref, b1_ref, w2_ref,
                   b2_ref, ws_ref, wd_ref, o_ref, os_ref, od_ref):
    x = x_ref[...]
    agg = pa_ref[0] + pa_ref[1] + pb_ref[0] + pb_ref[1]
    t = _silu(_dot(x, wx_ref[...]) + _dot(agg, wa_ref[...]) + b1_ref[...])
    xn = x + _dot(t, w2_ref[...]) + b2_ref[...]
    o_ref[...] = xn
    os_ref[...] = _dot(xn, ws_ref[...])
    od_ref[...] = _dot(xn, wd_ref[...])


def _node_last_body(x_ref, pa_ref, pb_ref, wx_ref, wa_ref, b1_ref, w2_ref,
                    b2_ref, d1_ref, db1_ref, d2_ref, db2_ref, o_ref):
    x = x_ref[...]
    agg = pa_ref[0] + pa_ref[1] + pb_ref[0] + pb_ref[1]
    t = _silu(_dot(x, wx_ref[...]) + _dot(agg, wa_ref[...]) + b1_ref[...])
    xn = x + _dot(t, w2_ref[...]) + b2_ref[...]
    u = _silu(_dot(xn, d1_ref[...]) + db1_ref[...])
    o_ref[...] = _dot(u, d2_ref[...]) + db2_ref[...]


def _row_block(bn):
    return pl.BlockSpec((bn, H), lambda i: (i, 0))


def _full(shape):
    return pl.BlockSpec(shape, lambda i: tuple(0 for _ in shape))


def _embed(h, w, b, ws, wd):
    return pl.pallas_call(
        _embed_body,
        grid=(N // BN,),
        in_specs=[pl.BlockSpec((BN, D), lambda i: (i, 0)),
                  _full((D, H)), _full((1, H)), _full((H, H)), _full((H, H))],
        out_specs=[_row_block(BN), _row_block(BN), _row_block(BN)],
        out_shape=[jax.ShapeDtypeStruct((N, H), _f32)] * 3,
    )(h, w, b.reshape(1, H), ws, wd)


def _edge_mlp(g, ea, wea, b1, w2, b2):
    return pl.pallas_call(
        _edge_body,
        grid=(ES // BE,),
        in_specs=[_row_block(BE),
                  pl.BlockSpec((BE, DE), lambda i: (i, 0)),
                  _full((DE, H)), _full((1, H)), _full((H, H)), _full((1, H))],
        out_specs=_row_block(BE),
        out_shape=jax.ShapeDtypeStruct((ES, H), _f32),
    )(g, ea, wea, b1.reshape(1, H), w2, b2.reshape(1, H))


def _parts_specs():
    return [pl.BlockSpec((NC, BN, H), lambda i: (0, i, 0)),
            pl.BlockSpec((NC, BN, H), lambda i: (0, i, 0))]


def _node_mid(x, pa, pb, wx, wa, b1, w2, b2, ws, wd):
    return pl.pallas_call(
        _node_mid_body,
        grid=(N // BN,),
        in_specs=[_row_block(BN)] + _parts_specs() +
                 [_full((H, H)), _full((H, H)), _full((1, H)),
                  _full((H, H)), _full((1, H)), _full((H, H)), _full((H, H))],
        out_specs=[_row_block(BN), _row_block(BN), _row_block(BN)],
        out_shape=[jax.ShapeDtypeStruct((N, H), _f32)] * 3,
    )(x, pa, pb, wx, wa, b1.reshape(1, H), w2, b2.reshape(1, H), ws, wd)


def _node_last(x, pa, pb, wx, wa, b1, w2, b2, d1W, d1b, d2W, d2b):
    d2p = jnp.zeros((H, 128), _f32).at[:, :3].set(d2W)
    db2p = jnp.zeros((1, 128), _f32).at[0, :3].set(d2b)
    y = pl.pallas_call(
        _node_last_body,
        grid=(N // BN,),
        in_specs=[_row_block(BN)] + _parts_specs() +
                 [_full((H, H)), _full((H, H)), _full((1, H)),
                  _full((H, H)), _full((1, H)), _full((H, H)), _full((1, H)),
                  _full((H, 128)), _full((1, 128))],
        out_specs=pl.BlockSpec((BN, 128), lambda i: (i, 0)),
        out_shape=jax.ShapeDtypeStruct((N, 128), _f32),
    )(x, pa, pb, wx, wa, b1.reshape(1, H), w2, b2.reshape(1, H),
      d1W, d1b.reshape(1, H), d2p, db2p)
    return y[:, :3]


def _worker_range(wid):
    c0 = wid * MAXC
    cnt = jnp.minimum(MAXC, NCHUNK - c0)
    return c0, cnt


def _sc_gather_body(xs_hbm, xd_hbm, row_hbm, col_hbm, g_hbm,
                    ridx, cidx, ident, b1a, b1b, b2a, b2b, shared,
                    s1a, s1b, s2a, s2b):
    cid = lax.axis_index("c")
    sid = lax.axis_index("s")
    wid = sid * NC + cid
    c0, cnt = _worker_range(wid)
    pltpu.sync_copy(row_hbm.at[pl.ds(c0, MAXC)], ridx)
    pltpu.sync_copy(col_hbm.at[pl.ds(c0, MAXC)], cidx)

    @pl.loop(0, CH // 16)
    def _(j):
        ia = lax.broadcasted_iota(_i32, (16,), 0) + j * 16
        ident[pl.ds(j * 16, 16)] = ia + sid * CH

    def fire(k, buf1, buf2, sem1, sem2):
        @pl.when(k < cnt)
        def _():
            pltpu.make_async_copy(xs_hbm.at[ridx.at[k]], buf1, sem1).start()
            pltpu.make_async_copy(xd_hbm.at[cidx.at[k]], buf2, sem2).start()

    def drain(k, buf1, buf2, sem1, sem2):
        @pl.when(k < cnt)
        def _():
            base = (c0 + k) * CH
            r0 = sid * CH
            pltpu.make_async_copy(xs_hbm.at[ridx.at[k]], buf1, sem1).wait()
            pltpu.make_async_copy(xd_hbm.at[cidx.at[k]], buf2, sem2).wait()
            pltpu.sync_copy(buf1, shared.at[pl.ds(r0, CH)])
            pltpu.sync_copy(buf2, shared.at[ident], add=True)
            pltpu.sync_copy(shared.at[pl.ds(r0, CH)], g_hbm.at[pl.ds(base, CH)])

    fire(0, b1a, b2a, s1a, s2a)

    @pl.loop(0, (MAXC + 1) // 2)
    def _(j):
        k = 2 * j
        fire(k + 1, b1b, b2b, s1b, s2b)
        drain(k, b1a, b2a, s1a, s2a)
        fire(k + 2, b1a, b2a, s1a, s2a)
        drain(k + 1, b1b, b2b, s1b, s2b)


def _sc_gather(xs, xd, row2d, col2d):
    k = pl.kernel(
        _sc_gather_body,
        out_type=jax.ShapeDtypeStruct((ES, H), _f32),
        mesh=_mesh,
        scratch_types=[pltpu.VMEM((MAXC, CH), _i32),
                       pltpu.VMEM((MAXC, CH), _i32),
                       pltpu.VMEM((CH,), _i32),
                       pltpu.VMEM((CH, H), _f32),
                       pltpu.VMEM((CH, H), _f32),
                       pltpu.VMEM((CH, H), _f32),
                       pltpu.VMEM((CH, H), _f32),
                       pltpu.VMEM_SHARED((NS * CH, H), _f32),
                       pltpu.SemaphoreType.DMA,
                       pltpu.SemaphoreType.DMA,
                       pltpu.SemaphoreType.DMA,
                       pltpu.SemaphoreType.DMA],
    )
    return k(xs, xd, row2d, col2d)


def _sc_scatter_body(m2_hbm, row_hbm, out_hbm, ridx, bufa, bufb, zbuf, acc,
                     sema, semb):
    cid = lax.axis_index("c")
    sid = lax.axis_index("s")
    wid = sid * NC + cid
    c0, cnt = _worker_range(wid)

    pltpu.sync_copy(row_hbm.at[pl.ds(c0, MAXC)], ridx)

    @pl.loop(0, ZR)
    def _(r):
        @pl.loop(0, H // 16)
        def _(j):
            zbuf[r, pl.ds(j * 16, 16)] = jnp.zeros((16,), _f32)

    @pl.loop(0, ZPW)
    def _(j):
        z = sid + j * NS

        @pl.when(z < NZC)
        def _():
            pltpu.sync_copy(zbuf, acc.at[pl.ds(z * ZR, ZR)])

    plsc.subcore_barrier()

    def fire(k, buf, sem):
        @pl.when(k < cnt)
        def _():
            base = (c0 + k) * CH
            pltpu.make_async_copy(m2_hbm.at[pl.ds(base, CH)], buf, sem).start()

    def drain(k, buf, sem):
        @pl.when(k < cnt)
        def _():
            base = (c0 + k) * CH
            pltpu.make_async_copy(m2_hbm.at[pl.ds(base, CH)], buf, sem).wait()
            pltpu.sync_copy(buf, acc.at[ridx.at[k]], add=True)

    fire(0, bufa, sema)

    @pl.loop(0, (MAXC + 1) // 2)
    def _(j):
        k = 2 * j
        fire(k + 1, bufb, semb)
        drain(k, bufa, sema)
        fire(k + 2, bufa, sema)
        drain(k + 1, bufb, semb)

    plsc.subcore_barrier()

    @pl.loop(0, ZPW)
    def _(j):
        z = sid + j * NS

        @pl.when(z < NZC)
        def _():
            r0 = z * ZR
            pltpu.sync_copy(acc.at[pl.ds(r0, ZR)],
                            out_hbm.at[cid, pl.ds(r0, ZR)])


def _sc_scatter(m2, row2d):
    k = pl.kernel(
        _sc_scatter_body,
        out_type=jax.ShapeDtypeStruct((NC, N, H), _f32),
        mesh=_mesh,
        scratch_types=[pltpu.VMEM((MAXC, CH), _i32),
                       pltpu.VMEM((CH, H), _f32),
                       pltpu.VMEM((CH, H), _f32),
                       pltpu.VMEM((ZR, H), _f32),
                       pltpu.VMEM_SHARED((N, H), _f32),
                       pltpu.SemaphoreType.DMA,
                       pltpu.SemaphoreType.DMA],
    )
    return k(m2, row2d)


def _slice_chunks(idx1d):
    c = idx1d.reshape(NSL, NCHUNK, CH)
    pad = ((0, NCPAD - NCHUNK), (0, 0))
    return [jnp.pad(c[s], pad) for s in range(NSL)]


def kernel(h, edges, edge_attr, emb_W, emb_b, e1W, e1b, e2W, e2b,
           n1W, n1b, n2W, n2b, d1W, d1b, d2W, d2b):
    rows = _slice_chunks(edges[0])
    cols = _slice_chunks(edges[1])
    eas = [edge_attr[s * ES:(s + 1) * ES] for s in range(NSL)]

    x, xs, xd = _embed(h, emb_W, emb_b, e1W[0, :H], e1W[0, H:2 * H])
    for i in range(NL):
        wea = e1W[i, 2 * H:]
        gs = [_sc_gather(xs, xd, rows[s], cols[s]) for s in range(NSL)]
        m2s = [_edge_mlp(gs[s], eas[s], wea, e1b[i], e2W[i], e2b[i])
               for s in range(NSL)]
        parts = [_sc_scatter(m2s[s], rows[s]) for s in range(NSL)]
        if i < NL - 1:
            x, xs, xd = _node_mid(x, parts[0], parts[1], n1W[i, :H],
                                  n1W[i, H:], n1b[i], n2W[i], n2b[i],
                                  e1W[i + 1, :H], e1W[i + 1, H:2 * H])
        else:
            return _node_last(x, parts[0], parts[1], n1W[i, :H], n1W[i, H:],
                              n1b[i], n2W[i], n2b[i], d1W, d1b, d2W, d2b)


# speedup vs baseline: 5.1137x; 1.0050x over previous
import jax
import jax.numpy as jnp
from jax import lax
from jax.experimental import pallas as pl
from jax.experimental.pallas import tpu as pltpu
from jax.experimental.pallas import tpu_sc as plsc

N = 10000
E = 320000
D = 128
H = 128
NL = 4
DE = 2

NSL = 2
ES = E // NSL

NC = 2
NS = 16
NW = NC * NS
CH = 80
NCHUNK = ES // CH
MAXC = 64
NCPAD = MAXC * NW

ZR = 80
NZC = N // ZR
ZPW = -(-NZC // NS)

BN = 2000
BE = 2000

_f32 = jnp.float32
_i32 = jnp.int32
_mesh = plsc.VectorSubcoreMesh(core_axis_name="c", subcore_axis_name="s")


def _dot(a, b):
    return jnp.dot(a, b, preferred_element_type=_f32)


def _silu(x):
    return x * (0.5 + 0.5 * jnp.tanh(0.5 * x))


def _embed_body(h_ref, w_ref, b_ref, ws_ref, wd_ref, o_ref, os_ref, od_ref):
    x = _dot(h_ref[...], w_ref[...]) + b_ref[...]
    o_ref[...] = x
    os_ref[...] = _dot(x, ws_ref[...])
    od_ref[...] = _dot(x, wd_ref[...])


def _edge_body(g_ref, ea_ref, wea_ref, b1_ref, w2_ref, b2_ref, o_ref):
    ea = ea_ref[...]
    pre = (g_ref[...] + b1_ref[...]
           + ea[:, 0:1] * wea_ref[0:1, :] + ea[:, 1:2] * wea_ref[1:2, :])
    m = _silu(pre)
    o_ref[...] = _silu(_dot(m, w2_ref[...]) + b2_ref[...])


def _node_mid_body(x_ref, pa_ref, pb_ref, wx_ref, wa_ref, b1_ref, w2_ref,
                   b2_ref, ws_ref, wd_ref, o_ref, os_ref, od_ref):
    x = x_ref[...]
    agg = pa_ref[0] + pa_ref[1] + pb_ref[0] + pb_ref[1]
    t = _silu(_dot(x, wx_ref[...]) + _dot(agg, wa_ref[...]) + b1_ref[...])
    xn = x + _dot(t, w2_ref[...]) + b2_ref[...]
    o_ref[...] = xn
    os_ref[...] = _dot(xn, ws_ref[...])
    od_ref[...] = _dot(xn, wd_ref[...])


def _node_last_body(x_ref, pa_ref, pb_ref, wx_ref, wa_ref, b1_ref, w2_ref,
                    b2_ref, d1_ref, db1_ref, d2_ref, db2_ref, o_ref):
    x = x_ref[...]
    agg = pa_ref[0] + pa_ref[1] + pb_ref[0] + pb_ref[1]
    t = _silu(_dot(x, wx_ref[...]) + _dot(agg, wa_ref[...]) + b1_ref[...])
    xn = x + _dot(t, w2_ref[...]) + b2_ref[...]
    u = _silu(_dot(xn, d1_ref[...]) + db1_ref[...])
    o_ref[...] = _dot(u, d2_ref[...]) + db2_ref[...]


def _row_block(bn):
    return pl.BlockSpec((bn, H), lambda i: (i, 0))


def _full(shape):
    return pl.BlockSpec(shape, lambda i: tuple(0 for _ in shape))


def _embed(h, w, b, ws, wd):
    return pl.pallas_call(
        _embed_body,
        grid=(N // BN,),
        in_specs=[pl.BlockSpec((BN, D), lambda i: (i, 0)),
                  _full((D, H)), _full((1, H)), _full((H, H)), _full((H, H))],
        out_specs=[_row_block(BN), _row_block(BN), _row_block(BN)],
        out_shape=[jax.ShapeDtypeStruct((N, H), _f32)] * 3,
    )(h, w, b.reshape(1, H), ws, wd)


def _edge_mlp(g, ea, wea, b1, w2, b2):
    return pl.pallas_call(
        _edge_body,
        grid=(ES // BE,),
        in_specs=[_row_block(BE),
                  pl.BlockSpec((BE, DE), lambda i: (i, 0)),
                  _full((DE, H)), _full((1, H)), _full((H, H)), _full((1, H))],
        out_specs=_row_block(BE),
        out_shape=jax.ShapeDtypeStruct((ES, H), _f32),
    )(g, ea, wea, b1.reshape(1, H), w2, b2.reshape(1, H))


def _parts_specs():
    return [pl.BlockSpec((NC, BN, H), lambda i: (0, i, 0)),
            pl.BlockSpec((NC, BN, H), lambda i: (0, i, 0))]


def _node_mid(x, pa, pb, wx, wa, b1, w2, b2, ws, wd):
    return pl.pallas_call(
        _node_mid_body,
        grid=(N // BN,),
        in_specs=[_row_block(BN)] + _parts_specs() +
                 [_full((H, H)), _full((H, H)), _full((1, H)),
                  _full((H, H)), _full((1, H)), _full((H, H)), _full((H, H))],
        out_specs=[_row_block(BN), _row_block(BN), _row_block(BN)],
        out_shape=[jax.ShapeDtypeStruct((N, H), _f32)] * 3,
    )(x, pa, pb, wx, wa, b1.reshape(1, H), w2, b2.reshape(1, H), ws, wd)


def _node_last(x, pa, pb, wx, wa, b1, w2, b2, d1W, d1b, d2W, d2b):
    d2p = jnp.zeros((H, 128), _f32).at[:, :3].set(d2W)
    db2p = jnp.zeros((1, 128), _f32).at[0, :3].set(d2b)
    y = pl.pallas_call(
        _node_last_body,
        grid=(N // BN,),
        in_specs=[_row_block(BN)] + _parts_specs() +
                 [_full((H, H)), _full((H, H)), _full((1, H)),
                  _full((H, H)), _full((1, H)), _full((H, H)), _full((1, H)),
                  _full((H, 128)), _full((1, 128))],
        out_specs=pl.BlockSpec((BN, 128), lambda i: (i, 0)),
        out_shape=jax.ShapeDtypeStruct((N, 128), _f32),
    )(x, pa, pb, wx, wa, b1.reshape(1, H), w2, b2.reshape(1, H),
      d1W, d1b.reshape(1, H), d2p, db2p)
    return y[:, :3]


def _worker_range(wid):
    c0 = wid * MAXC
    cnt = jnp.minimum(MAXC, NCHUNK - c0)
    return c0, cnt


def _sc_gather_body(xs_hbm, xd_hbm, row_hbm, col_hbm, g_hbm,
                    ridx, cidx, ident, b1a, b1b, b2a, b2b, shared,
                    s1a, s1b, s2a, s2b, sw):
    cid = lax.axis_index("c")
    sid = lax.axis_index("s")
    wid = sid * NC + cid
    c0, cnt = _worker_range(wid)
    pltpu.sync_copy(row_hbm.at[pl.ds(c0, MAXC)], ridx)
    pltpu.sync_copy(col_hbm.at[pl.ds(c0, MAXC)], cidx)

    @pl.loop(0, CH // 16)
    def _(j):
        ia = lax.broadcasted_iota(_i32, (16,), 0) + j * 16
        ident[pl.ds(j * 16, 16)] = ia + sid * CH

    def fire(k, buf1, buf2, sem1, sem2):
        @pl.when(k < cnt)
        def _():
            pltpu.make_async_copy(xs_hbm.at[ridx.at[k]], buf1, sem1).start()
            pltpu.make_async_copy(xd_hbm.at[cidx.at[k]], buf2, sem2).start()

    def wb(k):
        base = (c0 + k) * CH
        return pltpu.make_async_copy(shared.at[pl.ds(sid * CH, CH)],
                                     g_hbm.at[pl.ds(base, CH)], sw)

    def drain(k, buf1, buf2, sem1, sem2):
        @pl.when(k < cnt)
        def _():
            r0 = sid * CH
            pltpu.make_async_copy(xs_hbm.at[ridx.at[k]], buf1, sem1).wait()
            pltpu.make_async_copy(xd_hbm.at[cidx.at[k]], buf2, sem2).wait()

            @pl.when(k > 0)
            def _():
                wb(k - 1).wait()

            pltpu.sync_copy(buf1, shared.at[pl.ds(r0, CH)])
            pltpu.sync_copy(buf2, shared.at[ident], add=True)
            wb(k).start()

    fire(0, b1a, b2a, s1a, s2a)

    @pl.loop(0, (MAXC + 1) // 2)
    def _(j):
        k = 2 * j
        fire(k + 1, b1b, b2b, s1b, s2b)
        drain(k, b1a, b2a, s1a, s2a)
        fire(k + 2, b1a, b2a, s1a, s2a)
        drain(k + 1, b1b, b2b, s1b, s2b)

    @pl.when(cnt > 0)
    def _():
        wb(cnt - 1).wait()


def _sc_gather(xs, xd, row2d, col2d):
    k = pl.kernel(
        _sc_gather_body,
        out_type=jax.ShapeDtypeStruct((ES, H), _f32),
        mesh=_mesh,
        scratch_types=[pltpu.VMEM((MAXC, CH), _i32),
                       pltpu.VMEM((MAXC, CH), _i32),
                       pltpu.VMEM((CH,), _i32),
                       pltpu.VMEM((CH, H), _f32),
                       pltpu.VMEM((CH, H), _f32),
                       pltpu.VMEM((CH, H), _f32),
                       pltpu.VMEM((CH, H), _f32),
                       pltpu.VMEM_SHARED((NS * CH, H), _f32),
                       pltpu.SemaphoreType.DMA,
                       pltpu.SemaphoreType.DMA,
                       pltpu.SemaphoreType.DMA,
                       pltpu.SemaphoreType.DMA,
                       pltpu.SemaphoreType.DMA],
    )
    return k(xs, xd, row2d, col2d)


def _sc_scatter_body(m2_hbm, row_hbm, out_hbm, ridx, bufa, bufb, zbuf, acc,
                     sema, semb, semz):
    cid = lax.axis_index("c")
    sid = lax.axis_index("s")
    wid = sid * NC + cid
    c0, cnt = _worker_range(wid)

    pltpu.sync_copy(row_hbm.at[pl.ds(c0, MAXC)], ridx)

    @pl.loop(0, ZR)
    def _(r):
        @pl.loop(0, H // 16)
        def _(j):
            zbuf[r, pl.ds(j * 16, 16)] = jnp.zeros((16,), _f32)

    def zcopy(j):
        z = sid + j * NS
        return z < NZC, pltpu.make_async_copy(zbuf, acc.at[pl.ds(z * ZR, ZR)],
                                              semz)

    @pl.loop(0, ZPW)
    def _(j):
        ok, cp = zcopy(j)

        @pl.when(ok)
        def _():
            cp.start()

    @pl.loop(0, ZPW)
    def _(j):
        ok, cp = zcopy(j)

        @pl.when(ok)
        def _():
            cp.wait()

    plsc.subcore_barrier()

    def fire(k, buf, sem):
        @pl.when(k < cnt)
        def _():
            base = (c0 + k) * CH
            pltpu.make_async_copy(m2_hbm.at[pl.ds(base, CH)], buf, sem).start()

    def drain(k, buf, sem):
        @pl.when(k < cnt)
        def _():
            base = (c0 + k) * CH
            pltpu.make_async_copy(m2_hbm.at[pl.ds(base, CH)], buf, sem).wait()
            pltpu.sync_copy(buf, acc.at[ridx.at[k]], add=True)

    fire(0, bufa, sema)

    @pl.loop(0, (MAXC + 1) // 2)
    def _(j):
        k = 2 * j
        fire(k + 1, bufb, semb)
        drain(k, bufa, sema)
        fire(k + 2, bufa, sema)
        drain(k + 1, bufb, semb)

    plsc.subcore_barrier()

    def ocopy(j):
        z = sid + j * NS
        r0 = z * ZR
        return z < NZC, pltpu.make_async_copy(
            acc.at[pl.ds(r0, ZR)], out_hbm.at[cid, pl.ds(r0, ZR)], semz)

    @pl.loop(0, ZPW)
    def _(j):
        ok, cp = ocopy(j)

        @pl.when(ok)
        def _():
            cp.start()

    @pl.loop(0, ZPW)
    def _(j):
        ok, cp = ocopy(j)

        @pl.when(ok)
        def _():
            cp.wait()


def _sc_scatter(m2, row2d):
    k = pl.kernel(
        _sc_scatter_body,
        out_type=jax.ShapeDtypeStruct((NC, N, H), _f32),
        mesh=_mesh,
        scratch_types=[pltpu.VMEM((MAXC, CH), _i32),
                       pltpu.VMEM((CH, H), _f32),
                       pltpu.VMEM((CH, H), _f32),
                       pltpu.VMEM((ZR, H), _f32),
                       pltpu.VMEM_SHARED((N, H), _f32),
                       pltpu.SemaphoreType.DMA,
                       pltpu.SemaphoreType.DMA,
                       pltpu.SemaphoreType.DMA],
    )
    return k(m2, row2d)


def _slice_chunks(idx1d):
    c = idx1d.reshape(NSL, NCHUNK, CH)
    pad = ((0, NCPAD - NCHUNK), (0, 0))
    return [jnp.pad(c[s], pad) for s in range(NSL)]


def kernel(h, edges, edge_attr, emb_W, emb_b, e1W, e1b, e2W, e2b,
           n1W, n1b, n2W, n2b, d1W, d1b, d2W, d2b):
    rows = _slice_chunks(edges[0])
    cols = _slice_chunks(edges[1])
    eas = [edge_attr[s * ES:(s + 1) * ES] for s in range(NSL)]

    x, xs, xd = _embed(h, emb_W, emb_b, e1W[0, :H], e1W[0, H:2 * H])
    for i in range(NL):
        wea = e1W[i, 2 * H:]
        gs = [_sc_gather(xs, xd, rows[s], cols[s]) for s in range(NSL)]
        m2s = [_edge_mlp(gs[s], eas[s], wea, e1b[i], e2W[i], e2b[i])
               for s in range(NSL)]
        parts = [_sc_scatter(m2s[s], rows[s]) for s in range(NSL)]
        if i < NL - 1:
            x, xs, xd = _node_mid(x, parts[0], parts[1], n1W[i, :H],
                                  n1W[i, H:], n1b[i], n2W[i], n2b[i],
                                  e1W[i + 1, :H], e1W[i + 1, H:2 * H])
        else:
            return _node_last(x, parts[0], parts[1], n1W[i, :H], n1W[i, H:],
                              n1b[i], n2W[i], n2b[i], d1W, d1b, d2W, d2b)
